# Initial kernel scaffold; baseline (speedup 1.0000x reference)
import jax
import jax.numpy as jnp
from jax.experimental import pallas as pl


def kernel(continuous, down_idx, form_idx, pers_idx, def_idx, sit_idx, E_down, E_form, E_pers, E_def, E_sit, W1, b1, W2, b2, P1, b3, P2, b4):
    raise NotImplementedError("write your pallas kernel here")


# speedup vs baseline: 7.0586x; 7.0586x over previous
import jax
import jax.numpy as jnp
from jax.experimental import pallas as pl

B = 16384
HID = 256
OUT = 256
BB = 1024
NB = B // BB

_OFF = (0, 8, 40, 72, 136, 152, 280)
_NROWS = (5, 50, 50, 40, 4)


def _body(cont_ref, idx_ref, ed_ref, ef_ref, ep_ref, edf_ref, es_ref,
          w1_ref, b1_ref, w2_ref, b2_ref, p1_ref, b3_ref, p2_ref, b4_ref,
          out_ref):
    cont = cont_ref[...]
    idx = idx_ref[0]
    p1 = p1_ref[...]

    def onehot(row, n):
        r = idx[row]
        return (r[:, None] == jax.lax.broadcasted_iota(jnp.int32, (BB, n), 1)
                ).astype(jnp.float32)

    h = jnp.maximum(cont @ w1_ref[...] + b1_ref[...], 0.0)
    ce = h @ w2_ref[...] + b2_ref[...]

    acc = ce @ p1[_OFF[5]:_OFF[6]]
    tables = (ed_ref, ef_ref, ep_ref, edf_ref, es_ref)
    for f in range(5):
        g = tables[f][...] @ p1[_OFF[f]:_OFF[f + 1]]
        acc = acc + onehot(f, _NROWS[f]) @ g
    acc = acc + b3_ref[...]
    out_ref[...] = jnp.maximum(acc, 0.0) @ p2_ref[...] + b4_ref[...]


def kernel(continuous, down_idx, form_idx, pers_idx, def_idx, sit_idx,
           E_down, E_form, E_pers, E_def, E_sit,
           W1, b1, W2, b2, P1, b3, P2, b4):
    idx3d = jnp.stack([down_idx, form_idx, pers_idx, def_idx, sit_idx]
                      ).reshape(5, NB, BB).transpose(1, 0, 2)

    full = lambda shape: pl.BlockSpec(shape, lambda i: (0,) * len(shape))
    grid_spec = pl.GridSpec(
        grid=(NB,),
        in_specs=[
            pl.BlockSpec((BB, 3), lambda i: (i, 0)),
            pl.BlockSpec((1, 5, BB), lambda i: (i, 0, 0)),
            full((5, 8)), full((50, 32)), full((50, 32)), full((40, 64)),
            full((4, 16)),
            full((3, HID)), full((1, HID)),
            full((HID, HID // 2)), full((1, HID // 2)),
            full((280, HID)), full((1, HID)),
            full((HID, OUT)), full((1, OUT)),
        ],
        out_specs=pl.BlockSpec((BB, OUT), lambda i: (i, 0)),
    )
    return pl.pallas_call(
        _body,
        grid_spec=grid_spec,
        out_shape=jax.ShapeDtypeStruct((B, OUT), jnp.float32),
    )(continuous, idx3d, E_down, E_form, E_pers, E_def, E_sit,
      W1, b1.reshape(1, HID), W2, b2.reshape(1, HID // 2),
      P1, b3.reshape(1, HID), P2, b4.reshape(1, OUT))


# speedup vs baseline: 7.6016x; 1.0769x over previous
import jax
import jax.numpy as jnp
from jax.experimental import pallas as pl

B = 16384
HID = 256
OUT = 256
BB = 2048
NB = B // BB

_OFF = (0, 8, 40, 72, 136, 152, 280)
_NROWS = (5, 50, 50, 40, 4)


def _body(cont_ref, idx_ref, ed_ref, ef_ref, ep_ref, edf_ref, es_ref,
          w1_ref, b1_ref, w2_ref, b2_ref, p1_ref, b3_ref, p2_ref, b4_ref,
          out_ref):
    cont = cont_ref[...]
    idx = idx_ref[0]
    p1 = p1_ref[...]

    def onehot(row, n):
        r = idx[row]
        return (r[:, None] == jax.lax.broadcasted_iota(jnp.int32, (BB, n), 1)
                ).astype(jnp.float32)

    h = jnp.maximum(cont @ w1_ref[...] + b1_ref[...], 0.0)
    ce = h @ w2_ref[...] + b2_ref[...]

    acc = ce @ p1[_OFF[5]:_OFF[6]]
    tables = (ed_ref, ef_ref, ep_ref, edf_ref, es_ref)
    for f in range(5):
        g = tables[f][...] @ p1[_OFF[f]:_OFF[f + 1]]
        acc = acc + onehot(f, _NROWS[f]) @ g
    acc = acc + b3_ref[...]
    out_ref[...] = jnp.maximum(acc, 0.0) @ p2_ref[...] + b4_ref[...]


def kernel(continuous, down_idx, form_idx, pers_idx, def_idx, sit_idx,
           E_down, E_form, E_pers, E_def, E_sit,
           W1, b1, W2, b2, P1, b3, P2, b4):
    idx3d = jnp.stack([down_idx, form_idx, pers_idx, def_idx, sit_idx]
                      ).reshape(5, NB, BB).transpose(1, 0, 2)

    full = lambda shape: pl.BlockSpec(shape, lambda i: (0,) * len(shape))
    grid_spec = pl.GridSpec(
        grid=(NB,),
        in_specs=[
            pl.BlockSpec((BB, 3), lambda i: (i, 0)),
            pl.BlockSpec((1, 5, BB), lambda i: (i, 0, 0)),
            full((5, 8)), full((50, 32)), full((50, 32)), full((40, 64)),
            full((4, 16)),
            full((3, HID)), full((1, HID)),
            full((HID, HID // 2)), full((1, HID // 2)),
            full((280, HID)), full((1, HID)),
            full((HID, OUT)), full((1, OUT)),
        ],
        out_specs=pl.BlockSpec((BB, OUT), lambda i: (i, 0)),
    )
    return pl.pallas_call(
        _body,
        grid_spec=grid_spec,
        out_shape=jax.ShapeDtypeStruct((B, OUT), jnp.float32),
    )(continuous, idx3d, E_down, E_form, E_pers, E_def, E_sit,
      W1, b1.reshape(1, HID), W2, b2.reshape(1, HID // 2),
      P1, b3.reshape(1, HID), P2, b4.reshape(1, OUT))


# speedup vs baseline: 7.7597x; 1.0208x over previous
import jax
import jax.numpy as jnp
from jax.experimental import pallas as pl

B = 16384
HID = 256
OUT = 256
BB = 4096
NB = B // BB

_OFF = (0, 8, 40, 72, 136, 152, 280)
_NROWS = (5, 50, 50, 40, 4)


def _body(cont_ref, idx_ref, ed_ref, ef_ref, ep_ref, edf_ref, es_ref,
          w1_ref, b1_ref, w2_ref, b2_ref, p1_ref, b3_ref, p2_ref, b4_ref,
          out_ref):
    cont = cont_ref[...]
    idx = idx_ref[0]
    p1 = p1_ref[...]

    def onehot(row, n):
        r = idx[row]
        return (r[:, None] == jax.lax.broadcasted_iota(jnp.int32, (BB, n), 1)
                ).astype(jnp.float32)

    h = jnp.maximum(cont @ w1_ref[...] + b1_ref[...], 0.0)
    ce = h @ w2_ref[...] + b2_ref[...]

    acc = ce @ p1[_OFF[5]:_OFF[6]]
    tables = (ed_ref, ef_ref, ep_ref, edf_ref, es_ref)
    for f in range(5):
        g = tables[f][...] @ p1[_OFF[f]:_OFF[f + 1]]
        acc = acc + onehot(f, _NROWS[f]) @ g
    acc = acc + b3_ref[...]
    out_ref[...] = jnp.maximum(acc, 0.0) @ p2_ref[...] + b4_ref[...]


def kernel(continuous, down_idx, form_idx, pers_idx, def_idx, sit_idx,
           E_down, E_form, E_pers, E_def, E_sit,
           W1, b1, W2, b2, P1, b3, P2, b4):
    idx3d = jnp.stack([down_idx, form_idx, pers_idx, def_idx, sit_idx]
                      ).reshape(5, NB, BB).transpose(1, 0, 2)

    full = lambda shape: pl.BlockSpec(shape, lambda i: (0,) * len(shape))
    grid_spec = pl.GridSpec(
        grid=(NB,),
        in_specs=[
            pl.BlockSpec((BB, 3), lambda i: (i, 0)),
            pl.BlockSpec((1, 5, BB), lambda i: (i, 0, 0)),
            full((5, 8)), full((50, 32)), full((50, 32)), full((40, 64)),
            full((4, 16)),
            full((3, HID)), full((1, HID)),
            full((HID, HID // 2)), full((1, HID // 2)),
            full((280, HID)), full((1, HID)),
            full((HID, OUT)), full((1, OUT)),
        ],
        out_specs=pl.BlockSpec((BB, OUT), lambda i: (i, 0)),
    )
    return pl.pallas_call(
        _body,
        grid_spec=grid_spec,
        out_shape=jax.ShapeDtypeStruct((B, OUT), jnp.float32),
    )(continuous, idx3d, E_down, E_form, E_pers, E_def, E_sit,
      W1, b1.reshape(1, HID), W2, b2.reshape(1, HID // 2),
      P1, b3.reshape(1, HID), P2, b4.reshape(1, OUT))
